# baseline (speedup 1.0000x reference)
import functools

import jax
import jax.numpy as jnp
from jax import lax
from jax.experimental import pallas as pl
from jax.experimental.pallas import tpu as pltpu
from jax.experimental.pallas import tpu_sc as plsc

B, N, K, L = 8, 512, 2048, 128
NC, NS = 2, 16
BPC = B // NC
TPB = NS // BPC
RPT = K // TPB
SUB = 128
NSUB = RPT // SUB
ACC_ROWS = 2 * BPC * N
STRIPE = ACC_ROWS // NS

_mesh = plsc.VectorSubcoreMesh(core_axis_name="c", subcore_axis_name="s")


def _zero_buf(buf):
    zeros = jnp.zeros((16,), jnp.float32)

    def body(r, _):
        for col in range(L // 16):
            buf[r, pl.ds(col * 16, 16)] = zeros
        return 0

    lax.fori_loop(0, SUB, body, 0)


def _load_adjusted_idx(inds_hbm, idx_v, b, bl, k0):
    for m in range(2):
        for j in range(NSUB):
            pltpu.sync_copy(inds_hbm.at[m, b, pl.ds(k0 + j * SUB, SUB)],
                            idx_v.at[m * NSUB + j])
    for m in range(2):
        off = bl * N + m * (BPC * N)
        for j in range(NSUB):
            r = m * NSUB + j
            for col in range(SUB // 16):
                sl = pl.ds(col * 16, 16)
                idx_v[r, sl] = idx_v[r, sl] + off


def _stripe_io(tile, core):
    q = tile * STRIPE
    m = q // (BPC * N)
    rem = q % (BPC * N)
    g = m * (B * N) + core * (BPC * N) + rem
    return q, g


def _sc_scatter_body(pred_hbm, inds_hbm, out_hbm, acc_sh, idx_v, rows_v, zbuf):
    c = lax.axis_index("c")
    s = lax.axis_index("s")
    bl = s // TPB
    b = c * BPC + bl
    k0 = (s % TPB) * RPT

    _zero_buf(zbuf)
    q, g = _stripe_io(s, c)
    for t in range(STRIPE // SUB):
        pltpu.sync_copy(zbuf, acc_sh.at[pl.ds(q + t * SUB, SUB)])
    _load_adjusted_idx(inds_hbm, idx_v, b, bl, k0)
    plsc.subcore_barrier()

    for j in range(NSUB):
        pltpu.sync_copy(pred_hbm.at[b, pl.ds(k0 + j * SUB, SUB)], rows_v)
        pltpu.sync_copy(rows_v, acc_sh.at[idx_v.at[j]], add=True)
        pltpu.sync_copy(rows_v, acc_sh.at[idx_v.at[NSUB + j]], add=True)

    plsc.subcore_barrier()
    for t in range(STRIPE // SUB):
        pltpu.sync_copy(acc_sh.at[pl.ds(q + t * SUB, SUB)], rows_v)
        pltpu.sync_copy(rows_v, out_hbm.at[pl.ds(g + t * SUB, SUB)])


def _stage_a(a_hbm, a_sh, bufT, s, c):
    q, g = _stripe_io(s, c)
    for t in range(STRIPE // SUB):
        pltpu.sync_copy(a_hbm.at[pl.ds(g + t * SUB, SUB)], bufT)
        pltpu.sync_copy(bufT, a_sh.at[pl.ds(q + t * SUB, SUB)])


def _combine_relu(bufP, buf1, buf2):
    def body(r, _):
        for col in range(L // 16):
            sl = pl.ds(col * 16, 16)
            v = bufP[r, sl] + buf1[r, sl] + buf2[r, sl]
            bufP[r, sl] = jnp.maximum(v, 0.0)
        return 0

    lax.fori_loop(0, SUB, body, 0)


def _combine_relu_res(bufP, buf1, buf2, bufR):
    def body(r, _):
        for col in range(L // 16):
            sl = pl.ds(col * 16, 16)
            v = bufP[r, sl] + buf1[r, sl] + buf2[r, sl]
            bufP[r, sl] = jnp.maximum(v, 0.0) + bufR[r, sl]
        return 0

    lax.fori_loop(0, SUB, body, 0)


def _sc_gather_scatter_body(p_hbm, a_hbm, inds_hbm, newpred_hbm, s2_hbm,
                            a_sh, acc_sh, idx_v, bufP, buf1, buf2, sem):
    c = lax.axis_index("c")
    s = lax.axis_index("s")
    bl = s // TPB
    b = c * BPC + bl
    k0 = (s % TPB) * RPT

    _stage_a(a_hbm, a_sh, bufP, s, c)
    _zero_buf(bufP)
    q, g = _stripe_io(s, c)
    for t in range(STRIPE // SUB):
        pltpu.sync_copy(bufP, acc_sh.at[pl.ds(q + t * SUB, SUB)])
    _load_adjusted_idx(inds_hbm, idx_v, b, bl, k0)
    plsc.subcore_barrier()

    for j in range(NSUB):
        row0 = b * K + k0 + j * SUB
        pltpu.sync_copy(p_hbm.at[pl.ds(row0, SUB)], bufP)
        pltpu.async_copy(a_sh.at[idx_v.at[j]], buf1, sem).wait()
        pltpu.async_copy(a_sh.at[idx_v.at[NSUB + j]], buf2, sem).wait()
        _combine_relu(bufP, buf1, buf2)
        pltpu.sync_copy(bufP, newpred_hbm.at[pl.ds(row0, SUB)])
        pltpu.sync_copy(bufP, acc_sh.at[idx_v.at[j]], add=True)
        pltpu.sync_copy(bufP, acc_sh.at[idx_v.at[NSUB + j]], add=True)

    plsc.subcore_barrier()
    for t in range(STRIPE // SUB):
        pltpu.sync_copy(acc_sh.at[pl.ds(q + t * SUB, SUB)], bufP)
        pltpu.sync_copy(bufP, s2_hbm.at[pl.ds(g + t * SUB, SUB)])


def _sc_gather_out_body(p_hbm, a_hbm, inds_hbm, res_hbm, out_hbm,
                        a_sh, idx_v, bufP, buf1, buf2, bufR, sem):
    c = lax.axis_index("c")
    s = lax.axis_index("s")
    bl = s // TPB
    b = c * BPC + bl
    k0 = (s % TPB) * RPT

    _stage_a(a_hbm, a_sh, bufP, s, c)
    _load_adjusted_idx(inds_hbm, idx_v, b, bl, k0)
    plsc.subcore_barrier()

    for j in range(NSUB):
        row0 = b * K + k0 + j * SUB
        pltpu.sync_copy(p_hbm.at[pl.ds(row0, SUB)], bufP)
        pltpu.sync_copy(res_hbm.at[pl.ds(row0, SUB)], bufR)
        pltpu.async_copy(a_sh.at[idx_v.at[j]], buf1, sem).wait()
        pltpu.async_copy(a_sh.at[idx_v.at[NSUB + j]], buf2, sem).wait()
        _combine_relu_res(bufP, buf1, buf2, bufR)
        for cc in range(5):
            dst = (b * 5 + cc) * K + k0 + j * SUB
            pltpu.sync_copy(bufP, out_hbm.at[pl.ds(dst, SUB)])


_f32 = jnp.float32

_sc_scatter = pl.kernel(
    _sc_scatter_body, mesh=_mesh,
    out_type=[jax.ShapeDtypeStruct((2 * B * N, L), _f32)],
    scratch_types=[
        pltpu.VMEM_SHARED((ACC_ROWS, L), _f32),
        pltpu.VMEM((2 * NSUB, SUB), jnp.int32),
        pltpu.VMEM((SUB, L), _f32),
        pltpu.VMEM((SUB, L), _f32),
    ],
)

_sc_gather_scatter = pl.kernel(
    _sc_gather_scatter_body, mesh=_mesh,
    out_type=[jax.ShapeDtypeStruct((B * K, L), _f32),
              jax.ShapeDtypeStruct((2 * B * N, L), _f32)],
    scratch_types=[
        pltpu.VMEM_SHARED((ACC_ROWS, L), _f32),
        pltpu.VMEM_SHARED((ACC_ROWS, L), _f32),
        pltpu.VMEM((2 * NSUB, SUB), jnp.int32),
        pltpu.VMEM((SUB, L), _f32),
        pltpu.VMEM((SUB, L), _f32),
        pltpu.VMEM((SUB, L), _f32),
        pltpu.SemaphoreType.DMA,
    ],
)

_sc_gather_out = pl.kernel(
    _sc_gather_out_body, mesh=_mesh,
    out_type=[jax.ShapeDtypeStruct((B * 5 * K, L), _f32)],
    scratch_types=[
        pltpu.VMEM_SHARED((ACC_ROWS, L), _f32),
        pltpu.VMEM((2 * NSUB, SUB), jnp.int32),
        pltpu.VMEM((SUB, L), _f32),
        pltpu.VMEM((SUB, L), _f32),
        pltpu.VMEM((SUB, L), _f32),
        pltpu.VMEM((SUB, L), _f32),
        pltpu.SemaphoreType.DMA,
    ],
)


def _tc1_body(att_ref, pred_ref, s1_ref,
              w_obj_ref, w_ps_ref, w_po_ref, w_pred_ref, w_sp_ref, w_op_ref,
              nobj_ref, a_ref, p_ref):
    x = att_ref[0]
    mm = functools.partial(jnp.dot, preferred_element_type=_f32)
    nobj_ref[0] = jax.nn.relu(mm(x, w_obj_ref[0]) + mm(s1_ref[0, 0], w_ps_ref[0])
                              + mm(s1_ref[1, 0], w_po_ref[0]))
    a_ref[0, 0] = mm(x, w_sp_ref[0])
    a_ref[1, 0] = mm(x, w_op_ref[0])
    p_ref[0] = mm(pred_ref[0], w_pred_ref[0])


def _tc2_body(nobj1_ref, s2_ref, att_ref, npred1_ref,
              w_obj_ref, w_ps_ref, w_po_ref, w_pred_ref, w_sp_ref, w_op_ref,
              oobj_ref, a_ref, p_ref):
    x = nobj1_ref[0]
    mm = functools.partial(jnp.dot, preferred_element_type=_f32)
    obj2 = jax.nn.relu(mm(x, w_obj_ref[1]) + mm(s2_ref[0, 0], w_ps_ref[1])
                       + mm(s2_ref[1, 0], w_po_ref[1])) + att_ref[0]
    for cc in range(5):
        oobj_ref[0, cc] = obj2
    a_ref[0, 0] = mm(x, w_sp_ref[1])
    a_ref[1, 0] = mm(x, w_op_ref[1])
    p_ref[0] = mm(npred1_ref[0], w_pred_ref[1])


_w_spec = pl.BlockSpec((2, L, L), lambda i: (0, 0, 0))


def _tc1(att, pred, s1, *ws):
    return pl.pallas_call(
        _tc1_body,
        grid=(B,),
        in_specs=[
            pl.BlockSpec((1, N, L), lambda i: (i, 0, 0)),
            pl.BlockSpec((1, K, L), lambda i: (i, 0, 0)),
            pl.BlockSpec((2, 1, N, L), lambda i: (0, i, 0, 0)),
        ] + [_w_spec] * 6,
        out_specs=[
            pl.BlockSpec((1, N, L), lambda i: (i, 0, 0)),
            pl.BlockSpec((2, 1, N, L), lambda i: (0, i, 0, 0)),
            pl.BlockSpec((1, K, L), lambda i: (i, 0, 0)),
        ],
        out_shape=[
            jax.ShapeDtypeStruct((B, N, L), _f32),
            jax.ShapeDtypeStruct((2, B, N, L), _f32),
            jax.ShapeDtypeStruct((B, K, L), _f32),
        ],
    )(att, pred, s1, *ws)


def _tc2(nobj1, s2, att, npred1, *ws):
    return pl.pallas_call(
        _tc2_body,
        grid=(B,),
        in_specs=[
            pl.BlockSpec((1, N, L), lambda i: (i, 0, 0)),
            pl.BlockSpec((2, 1, N, L), lambda i: (0, i, 0, 0)),
            pl.BlockSpec((1, N, L), lambda i: (i, 0, 0)),
            pl.BlockSpec((1, K, L), lambda i: (i, 0, 0)),
        ] + [_w_spec] * 6,
        out_specs=[
            pl.BlockSpec((1, 5, N, L), lambda i: (i, 0, 0, 0)),
            pl.BlockSpec((2, 1, N, L), lambda i: (0, i, 0, 0)),
            pl.BlockSpec((1, K, L), lambda i: (i, 0, 0)),
        ],
        out_shape=[
            jax.ShapeDtypeStruct((B, 5, N, L), _f32),
            jax.ShapeDtypeStruct((2, B, N, L), _f32),
            jax.ShapeDtypeStruct((B, K, L), _f32),
        ],
    )(nobj1, s2, att, npred1, *ws)


def kernel(b, N_, K_, L_, att_feats, obj_dist, pred_fmap, rel_ind,
           W_obj, W_ps, W_po, W_pred, W_sp, W_op):
    del b, N_, K_, L_, obj_dist
    ws = (W_obj, W_ps, W_po, W_pred, W_sp, W_op)
    inds = jnp.transpose(rel_ind, (2, 0, 1))

    (s1,) = _sc_scatter(pred_fmap, inds)
    nobj1, a1, p1 = _tc1(att_feats, pred_fmap, s1.reshape(2, B, N, L), *ws)
    npred1, s2 = _sc_gather_scatter(p1.reshape(B * K, L),
                                    a1.reshape(2 * B * N, L), inds)
    oobj, a2, p2 = _tc2(nobj1, s2.reshape(2, B, N, L), att_feats,
                        npred1.reshape(B, K, L), *ws)
    (opred,) = _sc_gather_out(p2.reshape(B * K, L), a2.reshape(2 * B * N, L),
                              inds, pred_fmap.reshape(B * K, L))
    return (oobj.reshape(B * 5, N, L), opred.reshape(B * 5, K, L))


# speedup vs baseline: 1.1390x; 1.1390x over previous
import functools

import jax
import jax.numpy as jnp
from jax import lax
from jax.experimental import pallas as pl
from jax.experimental.pallas import tpu as pltpu
from jax.experimental.pallas import tpu_sc as plsc

B, N, K, L = 8, 512, 2048, 128
NC, NS = 2, 16
BPC = B // NC
TPB = NS // BPC
RPT = K // TPB
SUB = 64
NSUB = RPT // SUB
ACC_ROWS = 2 * BPC * N
STRIPE = ACC_ROWS // NS

_mesh = plsc.VectorSubcoreMesh(core_axis_name="c", subcore_axis_name="s")
_f32 = jnp.float32


def _zero_buf(buf):
    zeros = jnp.zeros((16,), jnp.float32)

    def body(r, _):
        for col in range(L // 16):
            buf[r, pl.ds(col * 16, 16)] = zeros
        return 0

    lax.fori_loop(0, SUB, body, 0)


def _load_adjusted_idx(inds_hbm, idx_v, b, k0, offs):
    for m in range(2):
        for j in range(NSUB):
            pltpu.sync_copy(inds_hbm.at[m, b, pl.ds(k0 + j * SUB, SUB)],
                            idx_v.at[m * NSUB + j])
    for m in range(2):
        for j in range(NSUB):
            r = m * NSUB + j
            for col in range(SUB // 16):
                sl = pl.ds(col * 16, 16)
                idx_v[r, sl] = idx_v[r, sl] + offs[m]


def _stripe_io(tile, core):
    q = tile * STRIPE
    m = q // (BPC * N)
    rem = q % (BPC * N)
    g = m * (B * N) + core * (BPC * N) + rem
    return q, g


def _tile_coords():
    c = lax.axis_index("c")
    s = lax.axis_index("s")
    bl = s // TPB
    b = c * BPC + bl
    k0 = (s % TPB) * RPT
    return c, s, bl, b, k0


def _sc_scatter_body(pred_hbm, inds_hbm, out_hbm,
                     acc_sh, idx_v, rows0, rows1, zbuf, semL):
    c, s, bl, b, k0 = _tile_coords()

    _zero_buf(zbuf)
    q, g = _stripe_io(s, c)
    for t in range(STRIPE // SUB):
        pltpu.sync_copy(zbuf, acc_sh.at[pl.ds(q + t * SUB, SUB)])
    _load_adjusted_idx(inds_hbm, idx_v, b, k0,
                       (bl * N, BPC * N + bl * N))
    plsc.subcore_barrier()

    bufs = (rows0, rows1)
    loads = [pltpu.async_copy(pred_hbm.at[b, pl.ds(k0, SUB)], bufs[0], semL)]
    for j in range(NSUB):
        buf = bufs[j % 2]
        loads[j].wait()
        if j + 1 < NSUB:
            loads.append(pltpu.async_copy(
                pred_hbm.at[b, pl.ds(k0 + (j + 1) * SUB, SUB)],
                bufs[(j + 1) % 2], semL))
        pltpu.sync_copy(buf, acc_sh.at[idx_v.at[j]], add=True)
        pltpu.sync_copy(buf, acc_sh.at[idx_v.at[NSUB + j]], add=True)

    plsc.subcore_barrier()
    for t in range(STRIPE // SUB):
        pltpu.sync_copy(acc_sh.at[pl.ds(q + t * SUB, SUB)], zbuf)
        pltpu.sync_copy(zbuf, out_hbm.at[pl.ds(g + t * SUB, SUB)])


def _combine_relu(bufP, buf1, buf2, bufR=None):
    def body(r, _):
        for col in range(L // 16):
            sl = pl.ds(col * 16, 16)
            v = bufP[r, sl] + buf1[r, sl] + buf2[r, sl]
            v = jnp.maximum(v, 0.0)
            if bufR is not None:
                v = v + bufR[r, sl]
            bufP[r, sl] = v
        return 0

    lax.fori_loop(0, SUB, body, 0)


def _sc_gather_scatter_body(p_hbm, a_hbm, inds_hbm, newpred_hbm, s2_hbm,
                            acc_sh, idx_v, gidx_v, p0, p1, g10, g11, g20, g21,
                            semP, semG, semW):
    c, s, bl, b, k0 = _tile_coords()

    _zero_buf(p0)
    q, g = _stripe_io(s, c)
    for t in range(STRIPE // SUB):
        pltpu.sync_copy(p0, acc_sh.at[pl.ds(q + t * SUB, SUB)])
    _load_adjusted_idx(inds_hbm, idx_v, b, k0,
                       (bl * N, BPC * N + bl * N))
    _load_adjusted_idx(inds_hbm, gidx_v, b, k0,
                       (b * N, B * N + b * N))
    plsc.subcore_barrier()

    bufsP, bufs1, bufs2 = (p0, p1), (g10, g11), (g20, g21)

    def start(j):
        row0 = b * K + k0 + j * SUB
        return (pltpu.async_copy(p_hbm.at[pl.ds(row0, SUB)], bufsP[j % 2], semP),
                pltpu.async_copy(a_hbm.at[gidx_v.at[j]], bufs1[j % 2], semG),
                pltpu.async_copy(a_hbm.at[gidx_v.at[NSUB + j]], bufs2[j % 2], semG))

    inflight = [start(0)]
    writes = []
    for j in range(NSUB):
        for h in inflight[j]:
            h.wait()
        if j + 1 < NSUB:
            inflight.append(start(j + 1))
        if j >= 2:
            for h in writes[j - 2]:
                h.wait()
        bufP = bufsP[j % 2]
        _combine_relu(bufP, bufs1[j % 2], bufs2[j % 2])
        row0 = b * K + k0 + j * SUB
        writes.append((
            pltpu.async_copy(bufP, newpred_hbm.at[pl.ds(row0, SUB)], semW),))
        pltpu.sync_copy(bufP, acc_sh.at[idx_v.at[j]], add=True)
        pltpu.sync_copy(bufP, acc_sh.at[idx_v.at[NSUB + j]], add=True)
    for ws in writes[max(0, NSUB - 2):]:
        for h in ws:
            h.wait()

    plsc.subcore_barrier()
    for t in range(STRIPE // SUB):
        pltpu.sync_copy(acc_sh.at[pl.ds(q + t * SUB, SUB)], p0)
        pltpu.sync_copy(p0, s2_hbm.at[pl.ds(g + t * SUB, SUB)])


def _sc_gather_out_body(p_hbm, a_hbm, inds_hbm, res_hbm, out_hbm,
                        idx_v, p0, p1, g10, g11, g20, g21, rbuf,
                        semP, semG, semW):
    c, s, bl, b, k0 = _tile_coords()

    _load_adjusted_idx(inds_hbm, idx_v, b, k0,
                       (b * N, B * N + b * N))

    bufsP, bufs1, bufs2 = (p0, p1), (g10, g11), (g20, g21)

    def start(j):
        row0 = b * K + k0 + j * SUB
        return (pltpu.async_copy(p_hbm.at[pl.ds(row0, SUB)], bufsP[j % 2], semP),
                pltpu.async_copy(a_hbm.at[idx_v.at[j]], bufs1[j % 2], semG),
                pltpu.async_copy(a_hbm.at[idx_v.at[NSUB + j]], bufs2[j % 2], semG))

    inflight = [start(0)]
    writes = []
    for j in range(NSUB):
        row0 = b * K + k0 + j * SUB
        pltpu.sync_copy(res_hbm.at[pl.ds(row0, SUB)], rbuf)
        for h in inflight[j]:
            h.wait()
        if j + 1 < NSUB:
            inflight.append(start(j + 1))
        if j >= 2:
            for h in writes[j - 2]:
                h.wait()
        bufP = bufsP[j % 2]
        _combine_relu(bufP, bufs1[j % 2], bufs2[j % 2], rbuf)
        writes.append(tuple(
            pltpu.async_copy(
                bufP, out_hbm.at[pl.ds((b * 5 + cc) * K + k0 + j * SUB, SUB)],
                semW)
            for cc in range(5)))
    for ws in writes[max(0, NSUB - 2):]:
        for h in ws:
            h.wait()


_sc_scatter = pl.kernel(
    _sc_scatter_body, mesh=_mesh,
    out_type=[jax.ShapeDtypeStruct((2 * B * N, L), _f32)],
    scratch_types=[
        pltpu.VMEM_SHARED((ACC_ROWS, L), _f32),
        pltpu.VMEM((2 * NSUB, SUB), jnp.int32),
        pltpu.VMEM((SUB, L), _f32),
        pltpu.VMEM((SUB, L), _f32),
        pltpu.VMEM((SUB, L), _f32),
        pltpu.SemaphoreType.DMA,
    ],
)

_sc_gather_scatter = pl.kernel(
    _sc_gather_scatter_body, mesh=_mesh,
    out_type=[jax.ShapeDtypeStruct((B * K, L), _f32),
              jax.ShapeDtypeStruct((2 * B * N, L), _f32)],
    scratch_types=[
        pltpu.VMEM_SHARED((ACC_ROWS, L), _f32),
        pltpu.VMEM((2 * NSUB, SUB), jnp.int32),
        pltpu.VMEM((2 * NSUB, SUB), jnp.int32),
        pltpu.VMEM((SUB, L), _f32),
        pltpu.VMEM((SUB, L), _f32),
        pltpu.VMEM((SUB, L), _f32),
        pltpu.VMEM((SUB, L), _f32),
        pltpu.VMEM((SUB, L), _f32),
        pltpu.VMEM((SUB, L), _f32),
        pltpu.SemaphoreType.DMA,
        pltpu.SemaphoreType.DMA,
        pltpu.SemaphoreType.DMA,
    ],
)

_sc_gather_out = pl.kernel(
    _sc_gather_out_body, mesh=_mesh,
    out_type=[jax.ShapeDtypeStruct((B * 5 * K, L), _f32)],
    scratch_types=[
        pltpu.VMEM((2 * NSUB, SUB), jnp.int32),
        pltpu.VMEM((SUB, L), _f32),
        pltpu.VMEM((SUB, L), _f32),
        pltpu.VMEM((SUB, L), _f32),
        pltpu.VMEM((SUB, L), _f32),
        pltpu.VMEM((SUB, L), _f32),
        pltpu.VMEM((SUB, L), _f32),
        pltpu.VMEM((SUB, L), _f32),
        pltpu.SemaphoreType.DMA,
        pltpu.SemaphoreType.DMA,
        pltpu.SemaphoreType.DMA,
    ],
)


def _tc1a_body(att_ref, pred_ref, w_pred_ref, w_sp_ref, w_op_ref,
               a_ref, p_ref):
    x = att_ref[0]
    mm = functools.partial(jnp.dot, preferred_element_type=_f32)
    a_ref[0, 0] = mm(x, w_sp_ref[0])
    a_ref[1, 0] = mm(x, w_op_ref[0])
    p_ref[0] = mm(pred_ref[0], w_pred_ref[0])


def _tc2_body(att_ref, s1_ref, s2_ref, npred1_ref,
              w_obj_ref, w_ps_ref, w_po_ref, w_pred_ref, w_sp_ref, w_op_ref,
              oobj_ref, a_ref, p_ref):
    att = att_ref[0]
    mm = functools.partial(jnp.dot, preferred_element_type=_f32)
    x1 = jax.nn.relu(mm(att, w_obj_ref[0]) + mm(s1_ref[0, 0], w_ps_ref[0])
                     + mm(s1_ref[1, 0], w_po_ref[0]))
    obj2 = jax.nn.relu(mm(x1, w_obj_ref[1]) + mm(s2_ref[0, 0], w_ps_ref[1])
                       + mm(s2_ref[1, 0], w_po_ref[1])) + att
    for cc in range(5):
        oobj_ref[0, cc] = obj2
    a_ref[0, 0] = mm(x1, w_sp_ref[1])
    a_ref[1, 0] = mm(x1, w_op_ref[1])
    p_ref[0] = mm(npred1_ref[0], w_pred_ref[1])


_w_spec = pl.BlockSpec((2, L, L), lambda i: (0, 0, 0))


def _tc1a(att, pred, w_pred, w_sp, w_op):
    return pl.pallas_call(
        _tc1a_body,
        grid=(B,),
        in_specs=[
            pl.BlockSpec((1, N, L), lambda i: (i, 0, 0)),
            pl.BlockSpec((1, K, L), lambda i: (i, 0, 0)),
        ] + [_w_spec] * 3,
        out_specs=[
            pl.BlockSpec((2, 1, N, L), lambda i: (0, i, 0, 0)),
            pl.BlockSpec((1, K, L), lambda i: (i, 0, 0)),
        ],
        out_shape=[
            jax.ShapeDtypeStruct((2, B, N, L), _f32),
            jax.ShapeDtypeStruct((B, K, L), _f32),
        ],
    )(att, pred, w_pred, w_sp, w_op)


def _tc2(att, s1, s2, npred1, *ws):
    return pl.pallas_call(
        _tc2_body,
        grid=(B,),
        in_specs=[
            pl.BlockSpec((1, N, L), lambda i: (i, 0, 0)),
            pl.BlockSpec((2, 1, N, L), lambda i: (0, i, 0, 0)),
            pl.BlockSpec((2, 1, N, L), lambda i: (0, i, 0, 0)),
            pl.BlockSpec((1, K, L), lambda i: (i, 0, 0)),
        ] + [_w_spec] * 6,
        out_specs=[
            pl.BlockSpec((1, 5, N, L), lambda i: (i, 0, 0, 0)),
            pl.BlockSpec((2, 1, N, L), lambda i: (0, i, 0, 0)),
            pl.BlockSpec((1, K, L), lambda i: (i, 0, 0)),
        ],
        out_shape=[
            jax.ShapeDtypeStruct((B, 5, N, L), _f32),
            jax.ShapeDtypeStruct((2, B, N, L), _f32),
            jax.ShapeDtypeStruct((B, K, L), _f32),
        ],
    )(att, s1, s2, npred1, *ws)


def kernel(b, N_, K_, L_, att_feats, obj_dist, pred_fmap, rel_ind,
           W_obj, W_ps, W_po, W_pred, W_sp, W_op):
    del b, N_, K_, L_, obj_dist
    ws = (W_obj, W_ps, W_po, W_pred, W_sp, W_op)
    inds = jnp.transpose(rel_ind, (2, 0, 1))

    (s1,) = _sc_scatter(pred_fmap, inds)
    a1, p1 = _tc1a(att_feats, pred_fmap, W_pred, W_sp, W_op)
    npred1, s2 = _sc_gather_scatter(p1.reshape(B * K, L),
                                    a1.reshape(2 * B * N, L), inds)
    oobj, a2, p2 = _tc2(att_feats, s1.reshape(2, B, N, L),
                        s2.reshape(2, B, N, L), npred1.reshape(B, K, L), *ws)
    (opred,) = _sc_gather_out(p2.reshape(B * K, L), a2.reshape(2 * B * N, L),
                              inds, pred_fmap.reshape(B * K, L))
    return (oobj.reshape(B * 5, N, L), opred.reshape(B * 5, K, L))


# speedup vs baseline: 1.2179x; 1.0692x over previous
import functools

import jax
import jax.numpy as jnp
from jax import lax
from jax.experimental import pallas as pl
from jax.experimental.pallas import tpu as pltpu
from jax.experimental.pallas import tpu_sc as plsc

B, N, K, L = 8, 512, 2048, 128
NC, NS = 2, 16
BPC = B // NC
TPB = NS // BPC
RPT = K // TPB
ACC_ROWS = 2 * BPC * N
STRIPE = ACC_ROWS // NS

_mesh = plsc.VectorSubcoreMesh(core_axis_name="c", subcore_axis_name="s")
_f32 = jnp.float32


def _zero_buf(buf, sub):
    zeros = jnp.zeros((16,), jnp.float32)

    def body(r, _):
        for col in range(L // 16):
            buf[r, pl.ds(col * 16, 16)] = zeros
        return 0

    lax.fori_loop(0, sub, body, 0)


def _load_idx(inds_hbm, idx_v, b, tk, offs, nsub, sub):
    k0 = tk * RPT
    for m in range(2):
        for j in range(nsub):
            pltpu.sync_copy(inds_hbm.at[m, b, pl.ds(k0 + j * sub, sub)],
                            idx_v.at[m, j])
    for m in range(2):
        for j in range(nsub):
            for col in range(sub // 16):
                sl = pl.ds(col * 16, 16)
                idx_v[m, j, sl] = idx_v[m, j, sl] + offs[m]


def _stripe_io(tile, core):
    q = tile * STRIPE
    m = q // (BPC * N)
    rem = q % (BPC * N)
    g = m * (B * N) + core * (BPC * N) + rem
    return q, g


def _tile_coords():
    c = lax.axis_index("c")
    s = lax.axis_index("s")
    bl = s // TPB
    b = c * BPC + bl
    tk = s % TPB
    k0 = tk * RPT
    return c, s, bl, b, tk, k0


def _init_acc(acc_sh, zsrc, q, sub, semZ):
    _zero_buf(zsrc, sub)
    hs = [pltpu.async_copy(zsrc, acc_sh.at[pl.ds(q + t * sub, sub)], semZ)
          for t in range(STRIPE // sub)]
    for h in hs:
        h.wait()


def _flush_acc(acc_sh, out_hbm, bufs, q, g, sub, semF):
    nt = STRIPE // sub
    hs = [None] * nt
    for t in range(nt):
        if t >= 2:
            hs[t - 2].wait()
        pltpu.sync_copy(acc_sh.at[pl.ds(q + t * sub, sub)], bufs[t % 2])
        hs[t] = pltpu.async_copy(bufs[t % 2],
                                 out_hbm.at[pl.ds(g + t * sub, sub)], semF)
    for t in range(max(0, nt - 2), nt):
        hs[t].wait()


S1_SUB = 128
S1_NSUB = RPT // S1_SUB


def _sc_scatter_body(pred_hbm, inds_hbm, out_hbm,
                     acc_sh, idx_v, rows0, rows1, semZ, semL, semA):
    c, s, bl, b, tk, k0 = _tile_coords()
    q, g = _stripe_io(s, c)
    sub, nsub = S1_SUB, S1_NSUB

    _init_acc(acc_sh, rows0, q, sub, semZ)
    _load_idx(inds_hbm, idx_v, b, tk,
              (bl * N, BPC * N + bl * N), nsub, sub)
    plsc.subcore_barrier()

    bufs = (rows0, rows1)
    loads = [pltpu.async_copy(pred_hbm.at[b, pl.ds(k0, sub)], bufs[0], semL)]
    for j in range(nsub):
        buf = bufs[j % 2]
        loads[j].wait()
        if j + 1 < nsub:
            loads.append(pltpu.async_copy(
                pred_hbm.at[b, pl.ds(k0 + (j + 1) * sub, sub)],
                bufs[(j + 1) % 2], semL))
        pltpu.sync_copy(buf, acc_sh.at[idx_v.at[0, j]], add=True)
        pltpu.sync_copy(buf, acc_sh.at[idx_v.at[1, j]], add=True)

    plsc.subcore_barrier()
    _flush_acc(acc_sh, out_hbm, bufs, q, g, sub, semL)


S2_SUB = 64
S2_NSUB = RPT // S2_SUB


def _combine_relu(bufP, buf1, buf2, sub, bufR=None):
    def body(r, _):
        for col in range(L // 16):
            sl = pl.ds(col * 16, 16)
            v = bufP[r, sl] + buf1[r, sl] + buf2[r, sl]
            v = jnp.maximum(v, 0.0)
            if bufR is not None:
                v = v + bufR[r, sl]
            bufP[r, sl] = v
        return 0

    lax.fori_loop(0, sub, body, 0)


def _sc_gather_scatter_body(p_hbm, a_hbm, inds_hbm, newpred_hbm, s2_hbm,
                            acc_sh, idx_v, gidx_v, p0, p1, g10, g11, g20, g21,
                            semZ, semP, semG, semW):
    c, s, bl, b, tk, k0 = _tile_coords()
    q, g = _stripe_io(s, c)
    sub, nsub = S2_SUB, S2_NSUB

    _init_acc(acc_sh, g10, q, sub, semZ)
    _load_idx(inds_hbm, idx_v, b, tk,
              (bl * N, BPC * N + bl * N), nsub, sub)
    _load_idx(inds_hbm, gidx_v, b, tk,
              (b * N, B * N + b * N), nsub, sub)
    plsc.subcore_barrier()

    bufsP, bufs1, bufs2 = (p0, p1), (g10, g11), (g20, g21)

    def start(j):
        row0 = b * K + k0 + j * sub
        return (pltpu.async_copy(p_hbm.at[pl.ds(row0, sub)], bufsP[j % 2], semP),
                pltpu.async_copy(a_hbm.at[gidx_v.at[0, j]], bufs1[j % 2], semG),
                pltpu.async_copy(a_hbm.at[gidx_v.at[1, j]], bufs2[j % 2], semG))

    inflight = [start(0)]
    writes = []
    for j in range(nsub):
        for h in inflight[j]:
            h.wait()
        if j >= 1:
            for h in writes[j - 1]:
                h.wait()
        if j + 1 < nsub:
            inflight.append(start(j + 1))
        bufP = bufsP[j % 2]
        _combine_relu(bufP, bufs1[j % 2], bufs2[j % 2], sub)
        row0 = b * K + k0 + j * sub
        writes.append((
            pltpu.async_copy(bufP, newpred_hbm.at[pl.ds(row0, sub)], semW),))
        pltpu.sync_copy(bufP, acc_sh.at[idx_v.at[0, j]], add=True)
        pltpu.sync_copy(bufP, acc_sh.at[idx_v.at[1, j]], add=True)
    for h in writes[nsub - 1]:
        h.wait()

    plsc.subcore_barrier()
    _flush_acc(acc_sh, s2_hbm, (p0, p1), q, g, sub, semP)


S3_SUB = 128
S3_NSUB = RPT // S3_SUB


def _sc_gather_out_body(p_hbm, a_hbm, inds_hbm, res_hbm, out_hbm,
                        idx_v, p0, p1, g10, g11, g20, g21, rbuf,
                        semP, semG, semW):
    c, s, bl, b, tk, k0 = _tile_coords()
    sub, nsub = S3_SUB, S3_NSUB

    _load_idx(inds_hbm, idx_v, b, tk,
              (b * N, B * N + b * N), nsub, sub)

    bufsP, bufs1, bufs2 = (p0, p1), (g10, g11), (g20, g21)

    def start(j):
        row0 = b * K + k0 + j * sub
        return (pltpu.async_copy(p_hbm.at[pl.ds(row0, sub)], bufsP[j % 2], semP),
                pltpu.async_copy(a_hbm.at[idx_v.at[0, j]], bufs1[j % 2], semG),
                pltpu.async_copy(a_hbm.at[idx_v.at[1, j]], bufs2[j % 2], semG))

    inflight = [start(0)]
    writes = []
    for j in range(nsub):
        row0 = b * K + k0 + j * sub
        pltpu.sync_copy(res_hbm.at[pl.ds(row0, sub)], rbuf)
        for h in inflight[j]:
            h.wait()
        if j >= 1:
            for h in writes[j - 1]:
                h.wait()
        if j + 1 < nsub:
            inflight.append(start(j + 1))
        bufP = bufsP[j % 2]
        _combine_relu(bufP, bufs1[j % 2], bufs2[j % 2], sub, rbuf)
        writes.append(tuple(
            pltpu.async_copy(
                bufP, out_hbm.at[pl.ds((b * 5 + cc) * K + k0 + j * sub, sub)],
                semW)
            for cc in range(5)))
    for h in writes[nsub - 1]:
        h.wait()


def _idx_t(nsub, sub):
    return pltpu.VMEM((2, nsub, sub), jnp.int32)


def _buf_t(sub):
    return pltpu.VMEM((sub, L), _f32)


_DMA = pltpu.SemaphoreType.DMA

_sc_scatter = pl.kernel(
    _sc_scatter_body, mesh=_mesh,
    out_type=[jax.ShapeDtypeStruct((2 * B * N, L), _f32)],
    scratch_types=[
        pltpu.VMEM_SHARED((ACC_ROWS, L), _f32),
        _idx_t(S1_NSUB, S1_SUB), _buf_t(S1_SUB), _buf_t(S1_SUB),
        _DMA, _DMA, _DMA,
    ],
)

_sc_gather_scatter = pl.kernel(
    _sc_gather_scatter_body, mesh=_mesh,
    out_type=[jax.ShapeDtypeStruct((B * K, L), _f32),
              jax.ShapeDtypeStruct((2 * B * N, L), _f32)],
    scratch_types=[
        pltpu.VMEM_SHARED((ACC_ROWS, L), _f32),
        _idx_t(S2_NSUB, S2_SUB), _idx_t(S2_NSUB, S2_SUB),
        _buf_t(S2_SUB), _buf_t(S2_SUB), _buf_t(S2_SUB),
        _buf_t(S2_SUB), _buf_t(S2_SUB), _buf_t(S2_SUB),
        _DMA, _DMA, _DMA, _DMA,
    ],
)

_sc_gather_out = pl.kernel(
    _sc_gather_out_body, mesh=_mesh,
    out_type=[jax.ShapeDtypeStruct((B * 5 * K, L), _f32)],
    scratch_types=[
        _idx_t(S3_NSUB, S3_SUB),
        _buf_t(S3_SUB), _buf_t(S3_SUB), _buf_t(S3_SUB),
        _buf_t(S3_SUB), _buf_t(S3_SUB), _buf_t(S3_SUB), _buf_t(S3_SUB),
        _DMA, _DMA, _DMA,
    ],
)


def _tc1a_body(att_ref, pred_ref, w_pred_ref, w_sp_ref, w_op_ref,
               a_ref, p_ref):
    x = att_ref[0]
    mm = functools.partial(jnp.dot, preferred_element_type=_f32)
    a_ref[0, 0] = mm(x, w_sp_ref[0])
    a_ref[1, 0] = mm(x, w_op_ref[0])
    p_ref[0] = mm(pred_ref[0], w_pred_ref[0])


def _tc2_body(att_ref, s1_ref, s2_ref, npred1_ref,
              w_obj_ref, w_ps_ref, w_po_ref, w_pred_ref, w_sp_ref, w_op_ref,
              oobj_ref, a_ref, p_ref):
    att = att_ref[0]
    mm = functools.partial(jnp.dot, preferred_element_type=_f32)
    x1 = jax.nn.relu(mm(att, w_obj_ref[0]) + mm(s1_ref[0, 0], w_ps_ref[0])
                     + mm(s1_ref[1, 0], w_po_ref[0]))
    obj2 = jax.nn.relu(mm(x1, w_obj_ref[1]) + mm(s2_ref[0, 0], w_ps_ref[1])
                       + mm(s2_ref[1, 0], w_po_ref[1])) + att
    for cc in range(5):
        oobj_ref[0, cc] = obj2
    a_ref[0, 0] = mm(x1, w_sp_ref[1])
    a_ref[1, 0] = mm(x1, w_op_ref[1])
    p_ref[0] = mm(npred1_ref[0], w_pred_ref[1])


_w_spec = pl.BlockSpec((2, L, L), lambda i: (0, 0, 0))


def _tc1a(att, pred, w_pred, w_sp, w_op):
    return pl.pallas_call(
        _tc1a_body,
        grid=(B,),
        in_specs=[
            pl.BlockSpec((1, N, L), lambda i: (i, 0, 0)),
            pl.BlockSpec((1, K, L), lambda i: (i, 0, 0)),
        ] + [_w_spec] * 3,
        out_specs=[
            pl.BlockSpec((2, 1, N, L), lambda i: (0, i, 0, 0)),
            pl.BlockSpec((1, K, L), lambda i: (i, 0, 0)),
        ],
        out_shape=[
            jax.ShapeDtypeStruct((2, B, N, L), _f32),
            jax.ShapeDtypeStruct((B, K, L), _f32),
        ],
    )(att, pred, w_pred, w_sp, w_op)


def _tc2(att, s1, s2, npred1, *ws):
    return pl.pallas_call(
        _tc2_body,
        grid=(B,),
        in_specs=[
            pl.BlockSpec((1, N, L), lambda i: (i, 0, 0)),
            pl.BlockSpec((2, 1, N, L), lambda i: (0, i, 0, 0)),
            pl.BlockSpec((2, 1, N, L), lambda i: (0, i, 0, 0)),
            pl.BlockSpec((1, K, L), lambda i: (i, 0, 0)),
        ] + [_w_spec] * 6,
        out_specs=[
            pl.BlockSpec((1, 5, N, L), lambda i: (i, 0, 0, 0)),
            pl.BlockSpec((2, 1, N, L), lambda i: (0, i, 0, 0)),
            pl.BlockSpec((1, K, L), lambda i: (i, 0, 0)),
        ],
        out_shape=[
            jax.ShapeDtypeStruct((B, 5, N, L), _f32),
            jax.ShapeDtypeStruct((2, B, N, L), _f32),
            jax.ShapeDtypeStruct((B, K, L), _f32),
        ],
    )(att, s1, s2, npred1, *ws)


def kernel(b, N_, K_, L_, att_feats, obj_dist, pred_fmap, rel_ind,
           W_obj, W_ps, W_po, W_pred, W_sp, W_op):
    del b, N_, K_, L_, obj_dist
    ws = (W_obj, W_ps, W_po, W_pred, W_sp, W_op)
    inds_t = jnp.transpose(rel_ind, (2, 0, 1))

    (s1,) = _sc_scatter(pred_fmap, inds_t)
    a1, p1 = _tc1a(att_feats, pred_fmap, W_pred, W_sp, W_op)
    npred1, s2 = _sc_gather_scatter(p1.reshape(B * K, L),
                                    a1.reshape(2 * B * N, L), inds_t)
    oobj, a2, p2 = _tc2(att_feats, s1.reshape(2, B, N, L),
                        s2.reshape(2, B, N, L), npred1.reshape(B, K, L), *ws)
    (opred,) = _sc_gather_out(p2.reshape(B * K, L), a2.reshape(2 * B * N, L),
                              inds_t, pred_fmap.reshape(B * K, L))
    return (oobj.reshape(B * 5, N, L), opred.reshape(B * 5, K, L))


# speedup vs baseline: 1.9141x; 1.5717x over previous
import functools

import jax
import jax.numpy as jnp
from jax import lax
from jax.experimental import pallas as pl
from jax.experimental.pallas import tpu as pltpu
from jax.experimental.pallas import tpu_sc as plsc

B, N, K, L = 8, 512, 2048, 128
NC, NS = 2, 16
BPC = B // NC
TPB = NS // BPC
RPT = K // TPB
ACC_ROWS = 2 * BPC * N
STRIPE = ACC_ROWS // NS
SUB = 128
NSUB = RPT // SUB

_mesh = plsc.VectorSubcoreMesh(core_axis_name="c", subcore_axis_name="s")
_f32 = jnp.float32


def _zero_buf(buf):
    zeros = jnp.zeros((16,), jnp.float32)

    def body(r, _):
        for col in range(L // 16):
            buf[r, pl.ds(col * 16, 16)] = zeros
        return 0

    lax.fori_loop(0, SUB, body, 0)


def _load_idx(inds_hbm, idx_v, b, k0, offs):
    for m in range(2):
        for j in range(NSUB):
            pltpu.sync_copy(inds_hbm.at[m, b, pl.ds(k0 + j * SUB, SUB)],
                            idx_v.at[m, j])
    for m in range(2):
        for j in range(NSUB):
            for col in range(SUB // 16):
                sl = pl.ds(col * 16, 16)
                idx_v[m, j, sl] = idx_v[m, j, sl] + offs[m]


def _sc_scatter_body(pred_hbm, inds_hbm, out_hbm,
                     acc_sh, idx_v, rows0, rows1, semZ, semL, semF):
    c = lax.axis_index("c")
    s = lax.axis_index("s")
    bl = s // TPB
    b = c * BPC + bl
    k0 = (s % TPB) * RPT
    q = s * STRIPE
    m = q // (BPC * N)
    g = m * (B * N) + c * (BPC * N) + q % (BPC * N)

    _zero_buf(rows0)
    zs = [pltpu.async_copy(rows0, acc_sh.at[pl.ds(q + t * SUB, SUB)], semZ)
          for t in range(STRIPE // SUB)]
    _load_idx(inds_hbm, idx_v, b, k0, (bl * N, BPC * N + bl * N))
    for h in zs:
        h.wait()
    plsc.subcore_barrier()

    bufs = (rows0, rows1)
    loads = [pltpu.async_copy(pred_hbm.at[b, pl.ds(k0, SUB)], bufs[0], semL)]
    for j in range(NSUB):
        buf = bufs[j % 2]
        loads[j].wait()
        if j + 1 < NSUB:
            loads.append(pltpu.async_copy(
                pred_hbm.at[b, pl.ds(k0 + (j + 1) * SUB, SUB)],
                bufs[(j + 1) % 2], semL))
        pltpu.sync_copy(buf, acc_sh.at[idx_v.at[0, j]], add=True)
        pltpu.sync_copy(buf, acc_sh.at[idx_v.at[1, j]], add=True)

    plsc.subcore_barrier()
    nt = STRIPE // SUB
    hs = [None] * nt
    for t in range(nt):
        if t >= 2:
            hs[t - 2].wait()
        pltpu.sync_copy(acc_sh.at[pl.ds(q + t * SUB, SUB)], bufs[t % 2])
        hs[t] = pltpu.async_copy(bufs[t % 2],
                                 out_hbm.at[pl.ds(g + t * SUB, SUB)], semF)
    for t in range(max(0, nt - 2), nt):
        hs[t].wait()


_DMA = pltpu.SemaphoreType.DMA

_sc_scatter = pl.kernel(
    _sc_scatter_body, mesh=_mesh,
    out_type=[jax.ShapeDtypeStruct((2 * B * N, L), _f32)],
    scratch_types=[
        pltpu.VMEM_SHARED((ACC_ROWS, L), _f32),
        pltpu.VMEM((2, NSUB, SUB), jnp.int32),
        pltpu.VMEM((SUB, L), _f32),
        pltpu.VMEM((SUB, L), _f32),
        _DMA, _DMA, _DMA,
    ],
)


def _onehots(ind_ref):
    ind_s = ind_ref[0, :, 0]
    ind_o = ind_ref[0, :, 1]
    iota_n = jax.lax.broadcasted_iota(jnp.int32, (K, N), 1)
    m_sT = (ind_s[:, None] == iota_n).astype(_f32)
    m_oT = (ind_o[:, None] == iota_n).astype(_f32)
    return m_sT, m_oT


_mm = functools.partial(jnp.dot, preferred_element_type=_f32)


def _tca_body(att_ref, pred_ref, ind_ref, w_pred_ref, w_sp_ref, w_op_ref,
              npred1_ref):
    x = att_ref[0]
    m_sT, m_oT = _onehots(ind_ref)
    a_s = _mm(x, w_sp_ref[0])
    a_o = _mm(x, w_op_ref[0])
    npred1_ref[0] = jax.nn.relu(_mm(pred_ref[0], w_pred_ref[0])
                                + _mm(m_sT, a_s) + _mm(m_oT, a_o))


def _tcb_body(att_ref, s1_ref, npred1_ref, pred_ref, ind_ref,
              w_obj_ref, w_ps_ref, w_po_ref, w_pred_ref, w_sp_ref, w_op_ref,
              opred_ref, x1_ref):
    att = att_ref[0]
    x1 = jax.nn.relu(_mm(att, w_obj_ref[0]) + _mm(s1_ref[0, 0], w_ps_ref[0])
                     + _mm(s1_ref[1, 0], w_po_ref[0]))
    m_sT, m_oT = _onehots(ind_ref)
    a2s = _mm(x1, w_sp_ref[1])
    a2o = _mm(x1, w_op_ref[1])
    npred2 = jax.nn.relu(_mm(npred1_ref[0], w_pred_ref[1])
                         + _mm(m_sT, a2s) + _mm(m_oT, a2o)) + pred_ref[0]
    for cc in range(5):
        opred_ref[0, cc] = npred2
    x1_ref[0] = x1


def _tcc_body(att_ref, x1_ref, s2_ref, w_obj_ref, w_ps_ref, w_po_ref,
              oobj_ref):
    obj2 = jax.nn.relu(_mm(x1_ref[0], w_obj_ref[1])
                       + _mm(s2_ref[0, 0], w_ps_ref[1])
                       + _mm(s2_ref[1, 0], w_po_ref[1])) + att_ref[0]
    for cc in range(5):
        oobj_ref[0, cc] = obj2


_w_spec = pl.BlockSpec((2, L, L), lambda i: (0, 0, 0))
_att_spec = pl.BlockSpec((1, N, L), lambda i: (i, 0, 0))
_pred_spec = pl.BlockSpec((1, K, L), lambda i: (i, 0, 0))
_ind_spec = pl.BlockSpec((1, K, 2), lambda i: (i, 0, 0))
_s_spec = pl.BlockSpec((2, 1, N, L), lambda i: (0, i, 0, 0))


def _tc_a(att, pred, rel_ind, w_pred, w_sp, w_op):
    return pl.pallas_call(
        _tca_body,
        grid=(B,),
        in_specs=[_att_spec, _pred_spec, _ind_spec] + [_w_spec] * 3,
        out_specs=[_pred_spec],
        out_shape=[jax.ShapeDtypeStruct((B, K, L), _f32)],
    )(att, pred, rel_ind, w_pred, w_sp, w_op)


def _tc_b(att, s1, npred1, pred, rel_ind, *ws):
    return pl.pallas_call(
        _tcb_body,
        grid=(B,),
        in_specs=[_att_spec, _s_spec, _pred_spec, _pred_spec, _ind_spec]
        + [_w_spec] * 6,
        out_specs=[
            pl.BlockSpec((1, 5, K, L), lambda i: (i, 0, 0, 0)),
            _att_spec,
        ],
        out_shape=[
            jax.ShapeDtypeStruct((B, 5, K, L), _f32),
            jax.ShapeDtypeStruct((B, N, L), _f32),
        ],
    )(att, s1, npred1, pred, rel_ind, *ws)


def _tc_c(att, x1, s2, w_obj, w_ps, w_po):
    return pl.pallas_call(
        _tcc_body,
        grid=(B,),
        in_specs=[_att_spec, _att_spec, _s_spec] + [_w_spec] * 3,
        out_specs=[pl.BlockSpec((1, 5, N, L), lambda i: (i, 0, 0, 0))],
        out_shape=[jax.ShapeDtypeStruct((B, 5, N, L), _f32)],
    )(att, x1, s2, w_obj, w_ps, w_po)


def kernel(b, N_, K_, L_, att_feats, obj_dist, pred_fmap, rel_ind,
           W_obj, W_ps, W_po, W_pred, W_sp, W_op):
    del b, N_, K_, L_, obj_dist
    ws = (W_obj, W_ps, W_po, W_pred, W_sp, W_op)
    inds_t = jnp.transpose(rel_ind, (2, 0, 1))

    (s1,) = _sc_scatter(pred_fmap, inds_t)
    (npred1,) = _tc_a(att_feats, pred_fmap, rel_ind, W_pred, W_sp, W_op)
    (s2,) = _sc_scatter(npred1, inds_t)
    opred, x1 = _tc_b(att_feats, s1.reshape(2, B, N, L), npred1,
                      pred_fmap, rel_ind, *ws)
    (oobj,) = _tc_c(att_feats, x1, s2.reshape(2, B, N, L), W_obj, W_ps, W_po)
    return (oobj.reshape(B * 5, N, L), opred.reshape(B * 5, K, L))
